# baseline (speedup 1.0000x reference)
import functools

import jax
import jax.numpy as jnp
from jax import lax
from jax.experimental import pallas as pl
from jax.experimental.pallas import tpu as pltpu
from jax.experimental.pallas import tpu_sc as plsc

_FF = 512
_EPB = 2


def _routing_body(T, E, K, NC, NS, idx_hbm, wgt_hbm, w_hbm, idx_v, wgt_v, wloc_v):
    NW = NC * NS
    tpw = T // NW
    vpw = tpw * K
    wid = lax.axis_index("s") * NC + lax.axis_index("c")
    pltpu.sync_copy(idx_hbm.at[pl.ds(wid * vpw, vpw)], idx_v)
    pltpu.sync_copy(wgt_hbm.at[pl.ds(wid * vpw, vpw)], wgt_v)
    for j in range(tpw * E // 16):
        wloc_v[pl.ds(j * 16, 16)] = jnp.zeros((16,), jnp.float32)
    lanes = lax.broadcasted_iota(jnp.int32, (16,), 0)
    local_t = lanes // K
    flat = local_t * E + idx_v[...]
    wv = wgt_v[...]
    for k in range(K):
        plsc.addupdate_scatter(wloc_v, [flat], wv, mask=(lanes % K) == k)
    pltpu.sync_copy(wloc_v, w_hbm.at[pl.ds(wid * tpw * E, tpw * E)])


def _routing_weights(top_k_indices, top_k_weights, E):
    T, K = top_k_indices.shape
    info = plsc.get_sparse_core_info()
    NC, NS = info.num_cores, info.num_subcores
    NW = NC * NS
    tpw = T // NW
    mesh = plsc.VectorSubcoreMesh(core_axis_name="c", subcore_axis_name="s",
                                  num_cores=NC, num_subcores=NS)
    k = pl.kernel(
        functools.partial(_routing_body, T, E, K, NC, NS),
        out_type=jax.ShapeDtypeStruct((T * E,), jnp.float32),
        mesh=mesh,
        scratch_types=[
            pltpu.VMEM((tpw * K,), jnp.int32),
            pltpu.VMEM((tpw * K,), jnp.float32),
            pltpu.VMEM((tpw * E,), jnp.float32),
        ],
        compiler_params=pltpu.CompilerParams(needs_layout_passes=False,
                                             skip_device_barrier=True),
    )
    w = k(top_k_indices.reshape(-1).astype(jnp.int32),
          top_k_weights.reshape(-1))
    return w.reshape(T, E)


def _moe_body(w_ref, hs_ref, gup_ref, down_ref, out_ref):
    g = pl.program_id(0)
    hs = hs_ref[...]
    W = w_ref[...]
    eidx = lax.broadcasted_iota(jnp.int32, W.shape, 1)
    contrib = jnp.zeros_like(out_ref)
    for i in range(_EPB):
        e = g * _EPB + i
        w = jnp.sum(jnp.where(eidx == e, W, 0.0), axis=1)
        gu = jax.lax.dot_general(
            hs, gup_ref[i], (((1,), (1,)), ((), ())),
            preferred_element_type=jnp.float32)
        gate = gu[:, :_FF]
        up = gu[:, _FF:]
        act = gate * jax.nn.sigmoid(gate) * up
        eo = jax.lax.dot_general(
            act, down_ref[i], (((1,), (1,)), ((), ())),
            preferred_element_type=jnp.float32)
        contrib = contrib + eo * w[:, None]

    @pl.when(g == 0)
    def _init():
        out_ref[...] = contrib

    @pl.when(g != 0)
    def _acc():
        out_ref[...] += contrib


def kernel(hidden_states, top_k_indices, top_k_weights, gate_up_proj, down_proj):
    T, H = hidden_states.shape
    E, FF2, _ = gate_up_proj.shape

    W = _routing_weights(top_k_indices, top_k_weights, E)

    return pl.pallas_call(
        _moe_body,
        grid=(E // _EPB,),
        in_specs=[
            pl.BlockSpec((T, E), lambda e: (0, 0)),
            pl.BlockSpec((T, H), lambda e: (0, 0)),
            pl.BlockSpec((_EPB, FF2, H), lambda e: (e, 0, 0)),
            pl.BlockSpec((_EPB, H, FF2 // 2), lambda e: (e, 0, 0)),
        ],
        out_specs=pl.BlockSpec((T, H), lambda e: (0, 0)),
        out_shape=jax.ShapeDtypeStruct((T, H), jnp.float32),
        compiler_params=pltpu.CompilerParams(
            dimension_semantics=("arbitrary",),
        ),
    )(W, hidden_states, gate_up_proj, down_proj)


# speedup vs baseline: 1.0175x; 1.0175x over previous
import functools

import jax
import jax.numpy as jnp
from jax import lax
from jax.experimental import pallas as pl
from jax.experimental.pallas import tpu as pltpu
from jax.experimental.pallas import tpu_sc as plsc

_FF = 512
_EPB = 2


def _routing_body(T, E, K, NC, NS, idx_hbm, wgt_hbm, w_hbm, idx_v, wgt_v, wloc_v):
    NW = NC * NS
    tpw = T // NW
    vpw = tpw * K
    wid = lax.axis_index("s") * NC + lax.axis_index("c")
    pltpu.sync_copy(idx_hbm.at[pl.ds(wid * vpw, vpw)], idx_v)
    pltpu.sync_copy(wgt_hbm.at[pl.ds(wid * vpw, vpw)], wgt_v)
    for j in range(tpw * E // 16):
        wloc_v[pl.ds(j * 16, 16)] = jnp.zeros((16,), jnp.float32)
    lanes = lax.broadcasted_iota(jnp.int32, (16,), 0)
    local_t = lanes // K
    flat = local_t * E + idx_v[...]
    wv = wgt_v[...]
    for k in range(K):
        plsc.addupdate_scatter(wloc_v, [flat], wv, mask=(lanes % K) == k)
    pltpu.sync_copy(wloc_v, w_hbm.at[pl.ds(wid * tpw * E, tpw * E)])


def _routing_weights(top_k_indices, top_k_weights, E):
    T, K = top_k_indices.shape
    info = plsc.get_sparse_core_info()
    NC, NS = info.num_cores, info.num_subcores
    NW = NC * NS
    tpw = T // NW
    mesh = plsc.VectorSubcoreMesh(core_axis_name="c", subcore_axis_name="s",
                                  num_cores=NC, num_subcores=NS)
    k = pl.kernel(
        functools.partial(_routing_body, T, E, K, NC, NS),
        out_type=jax.ShapeDtypeStruct((T * E,), jnp.float32),
        mesh=mesh,
        scratch_types=[
            pltpu.VMEM((tpw * K,), jnp.int32),
            pltpu.VMEM((tpw * K,), jnp.float32),
            pltpu.VMEM((tpw * E,), jnp.float32),
        ],
        compiler_params=pltpu.CompilerParams(needs_layout_passes=False),
    )
    w = k(top_k_indices.reshape(-1).astype(jnp.int32),
          top_k_weights.reshape(-1))
    return w.reshape(T, E)


def _moe_body(w_ref, hs_ref, gate_w_ref, up_w_ref, down_ref, out_ref):
    g = pl.program_id(0)
    hs = hs_ref[...]
    W = w_ref[...]
    eidx = lax.broadcasted_iota(jnp.int32, W.shape, 1)
    contrib = jnp.zeros_like(out_ref)
    for i in range(_EPB):
        e = g * _EPB + i
        w = jnp.sum(jnp.where(eidx == e, W, 0.0), axis=1)
        gate = jax.lax.dot_general(
            hs, gate_w_ref[i, 0], (((1,), (1,)), ((), ())),
            preferred_element_type=jnp.float32)
        up = jax.lax.dot_general(
            hs, up_w_ref[i, 0], (((1,), (1,)), ((), ())),
            preferred_element_type=jnp.float32)
        act = gate * jax.nn.sigmoid(gate) * up
        eo = jax.lax.dot_general(
            act, down_ref[i], (((1,), (1,)), ((), ())),
            preferred_element_type=jnp.float32)
        contrib = contrib + eo * w[:, None]

    @pl.when(g == 0)
    def _init():
        out_ref[...] = contrib

    @pl.when(g != 0)
    def _acc():
        out_ref[...] += contrib


def kernel(hidden_states, top_k_indices, top_k_weights, gate_up_proj, down_proj):
    T, H = hidden_states.shape
    E, FF2, _ = gate_up_proj.shape

    W = _routing_weights(top_k_indices, top_k_weights, E)
    gup4 = gate_up_proj.reshape(E, 2, FF2 // 2, H)

    return pl.pallas_call(
        _moe_body,
        grid=(E // _EPB,),
        in_specs=[
            pl.BlockSpec((T, E), lambda e: (0, 0)),
            pl.BlockSpec((T, H), lambda e: (0, 0)),
            pl.BlockSpec((_EPB, 1, FF2 // 2, H), lambda e: (e, 0, 0, 0)),
            pl.BlockSpec((_EPB, 1, FF2 // 2, H), lambda e: (e, 1, 0, 0)),
            pl.BlockSpec((_EPB, H, FF2 // 2), lambda e: (e, 0, 0)),
        ],
        out_specs=pl.BlockSpec((T, H), lambda e: (0, 0)),
        out_shape=jax.ShapeDtypeStruct((T, H), jnp.float32),
        compiler_params=pltpu.CompilerParams(
            dimension_semantics=("arbitrary",),
        ),
    )(W, hidden_states, gup4, gup4, down_proj)


# speedup vs baseline: 1.0203x; 1.0028x over previous
import functools

import jax
import jax.numpy as jnp
from jax import lax
from jax.experimental import pallas as pl
from jax.experimental.pallas import tpu as pltpu
from jax.experimental.pallas import tpu_sc as plsc

_FF = 512
_EPB = 2


def _routing_body(T, E, K, NC, NS, idx_hbm, wgt_hbm, w_hbm, idx_v, wgt_v, wloc_v):
    NW = NC * NS
    tpw = T // NW
    vpw = tpw * K
    wid = lax.axis_index("s") * NC + lax.axis_index("c")
    pltpu.sync_copy(idx_hbm.at[pl.ds(wid * vpw, vpw)], idx_v)
    pltpu.sync_copy(wgt_hbm.at[pl.ds(wid * vpw, vpw)], wgt_v)
    for j in range(tpw * E // 16):
        wloc_v[pl.ds(j * 16, 16)] = jnp.zeros((16,), jnp.float32)
    lanes = lax.broadcasted_iota(jnp.int32, (16,), 0)
    local_t = lanes // K
    flat = local_t * E + idx_v[...]
    wv = wgt_v[...]
    for k in range(K):
        plsc.addupdate_scatter(wloc_v, [flat], wv, mask=(lanes % K) == k)
    pltpu.sync_copy(wloc_v, w_hbm.at[pl.ds(wid * tpw * E, tpw * E)])


def _routing_weights(top_k_indices, top_k_weights, E):
    T, K = top_k_indices.shape
    info = plsc.get_sparse_core_info()
    NC, NS = info.num_cores, info.num_subcores
    NW = NC * NS
    tpw = T // NW
    mesh = plsc.VectorSubcoreMesh(core_axis_name="c", subcore_axis_name="s",
                                  num_cores=NC, num_subcores=NS)
    k = pl.kernel(
        functools.partial(_routing_body, T, E, K, NC, NS),
        out_type=jax.ShapeDtypeStruct((T * E,), jnp.float32),
        mesh=mesh,
        scratch_types=[
            pltpu.VMEM((tpw * K,), jnp.int32),
            pltpu.VMEM((tpw * K,), jnp.float32),
            pltpu.VMEM((tpw * E,), jnp.float32),
        ],
        compiler_params=pltpu.CompilerParams(needs_layout_passes=False),
    )
    w = k(top_k_indices.reshape(-1).astype(jnp.int32),
          top_k_weights.reshape(-1))
    return w.reshape(T, E)


def _ffn(hs, gup, down):
    gu = jax.lax.dot_general(
        hs, gup, (((1,), (1,)), ((), ())),
        preferred_element_type=jnp.float32)
    gate = gu[:, :_FF]
    up = gu[:, _FF:]
    act = gate * jax.nn.sigmoid(gate) * up
    return jax.lax.dot_general(
        act, down, (((1,), (1,)), ((), ())),
        preferred_element_type=jnp.float32)


def _moe_body_lo(idx_ref, wgt_ref, hs_ref, gup_ref, down_ref, out_ref):
    g = pl.program_id(0)
    hs = hs_ref[...]
    contrib = jnp.zeros_like(out_ref)
    for i in range(_EPB):
        e = g * _EPB + i
        w = jnp.sum(jnp.where(idx_ref[...] == e, wgt_ref[...], 0.0), axis=1)
        contrib = contrib + _ffn(hs, gup_ref[i], down_ref[i]) * w[:, None]

    @pl.when(g == 0)
    def _init():
        out_ref[...] = contrib

    @pl.when(g != 0)
    def _acc():
        out_ref[...] += contrib


def _moe_body_hi(e0, w_ref, hs_ref, gup_ref, down_ref, acc_ref, out_ref):
    g = pl.program_id(0)
    hs = hs_ref[...]
    W = w_ref[...]
    eidx = lax.broadcasted_iota(jnp.int32, W.shape, 1)
    contrib = jnp.zeros_like(out_ref)
    for i in range(_EPB):
        e = e0 + g * _EPB + i
        w = jnp.sum(jnp.where(eidx == e, W, 0.0), axis=1)
        contrib = contrib + _ffn(hs, gup_ref[i], down_ref[i]) * w[:, None]

    @pl.when(g == 0)
    def _init():
        out_ref[...] = acc_ref[...] + contrib

    @pl.when(g != 0)
    def _acc():
        out_ref[...] += contrib


def kernel(hidden_states, top_k_indices, top_k_weights, gate_up_proj, down_proj):
    T, H = hidden_states.shape
    E, FF2, _ = gate_up_proj.shape
    K = top_k_indices.shape[1]
    E_LO = E // 2
    G_HI = E_LO // _EPB

    W = _routing_weights(top_k_indices, top_k_weights, E)

    out_lo = pl.pallas_call(
        _moe_body_lo,
        grid=(E_LO // _EPB,),
        in_specs=[
            pl.BlockSpec((T, K), lambda e: (0, 0)),
            pl.BlockSpec((T, K), lambda e: (0, 0)),
            pl.BlockSpec((T, H), lambda e: (0, 0)),
            pl.BlockSpec((_EPB, FF2, H), lambda e: (e, 0, 0)),
            pl.BlockSpec((_EPB, H, FF2 // 2), lambda e: (e, 0, 0)),
        ],
        out_specs=pl.BlockSpec((T, H), lambda e: (0, 0)),
        out_shape=jax.ShapeDtypeStruct((T, H), jnp.float32),
        compiler_params=pltpu.CompilerParams(
            dimension_semantics=("arbitrary",),
        ),
    )(top_k_indices, top_k_weights, hidden_states, gate_up_proj, down_proj)

    return pl.pallas_call(
        functools.partial(_moe_body_hi, E_LO),
        grid=(G_HI,),
        in_specs=[
            pl.BlockSpec((T, E), lambda e: (0, 0)),
            pl.BlockSpec((T, H), lambda e: (0, 0)),
            pl.BlockSpec((_EPB, FF2, H), lambda e, _g=G_HI: (e + _g, 0, 0)),
            pl.BlockSpec((_EPB, H, FF2 // 2), lambda e, _g=G_HI: (e + _g, 0, 0)),
            pl.BlockSpec((T, H), lambda e: (0, 0)),
        ],
        out_specs=pl.BlockSpec((T, H), lambda e: (0, 0)),
        out_shape=jax.ShapeDtypeStruct((T, H), jnp.float32),
        compiler_params=pltpu.CompilerParams(
            dimension_semantics=("arbitrary",),
        ),
    )(W, hidden_states, gate_up_proj, down_proj, out_lo)


# speedup vs baseline: 1.0436x; 1.0228x over previous
import functools

import jax
import jax.numpy as jnp
from jax import lax
from jax.experimental import pallas as pl
from jax.experimental.pallas import tpu as pltpu
from jax.experimental.pallas import tpu_sc as plsc

_FF = 512
_EPB = 2


def _routing_body(T, E, K, NC, NS, idx_hbm, wgt_hbm, w_hbm, idx_v, wgt_v, wloc_v):
    NW = NC * NS
    tpw = T // NW
    vpw = tpw * K
    wid = lax.axis_index("s") * NC + lax.axis_index("c")
    pltpu.sync_copy(idx_hbm.at[pl.ds(wid * vpw, vpw)], idx_v)
    pltpu.sync_copy(wgt_hbm.at[pl.ds(wid * vpw, vpw)], wgt_v)
    for j in range(tpw * E // 16):
        wloc_v[pl.ds(j * 16, 16)] = jnp.zeros((16,), jnp.float32)
    lanes = lax.broadcasted_iota(jnp.int32, (16,), 0)
    local_t = lanes // K
    flat = local_t * E + idx_v[...]
    wv = wgt_v[...]
    for k in range(K):
        plsc.addupdate_scatter(wloc_v, [flat], wv, mask=(lanes % K) == k)
    pltpu.sync_copy(wloc_v, w_hbm.at[pl.ds(wid * tpw * E, tpw * E)])


def _routing_weights(top_k_indices, top_k_weights, E):
    T, K = top_k_indices.shape
    info = plsc.get_sparse_core_info()
    NC, NS = info.num_cores, info.num_subcores
    NW = NC * NS
    tpw = T // NW
    mesh = plsc.VectorSubcoreMesh(core_axis_name="c", subcore_axis_name="s",
                                  num_cores=NC, num_subcores=NS)
    k = pl.kernel(
        functools.partial(_routing_body, T, E, K, NC, NS),
        out_type=jax.ShapeDtypeStruct((T * E,), jnp.float32),
        mesh=mesh,
        scratch_types=[
            pltpu.VMEM((tpw * K,), jnp.int32),
            pltpu.VMEM((tpw * K,), jnp.float32),
            pltpu.VMEM((tpw * E,), jnp.float32),
        ],
        compiler_params=pltpu.CompilerParams(needs_layout_passes=False),
    )
    w = k(top_k_indices.reshape(-1).astype(jnp.int32),
          top_k_weights.reshape(-1))
    return w.reshape(T, E)


def _moe_body(w_ref, hs_ref, gup_ref, down_ref, out_ref):
    g = pl.program_id(0)
    hs = hs_ref[...]
    W = w_ref[...]
    eidx = lax.broadcasted_iota(jnp.int32, W.shape, 1)
    contrib = jnp.zeros_like(out_ref)
    for i in range(_EPB):
        e = g * _EPB + i
        w = jnp.sum(jnp.where(eidx == e, W, 0.0), axis=1)
        gu = jax.lax.dot_general(
            hs, gup_ref[i], (((1,), (1,)), ((), ())),
            preferred_element_type=jnp.float32)
        gate = gu[:, :_FF]
        up = gu[:, _FF:]
        act = gate * jax.nn.sigmoid(gate) * up
        eo = jax.lax.dot_general(
            act, down_ref[i], (((1,), (1,)), ((), ())),
            preferred_element_type=jnp.float32)
        contrib = contrib + eo * w[:, None]

    @pl.when(g == 0)
    def _init():
        out_ref[...] = contrib

    @pl.when(g != 0)
    def _acc():
        out_ref[...] += contrib


def kernel(hidden_states, top_k_indices, top_k_weights, gate_up_proj, down_proj):
    T, H = hidden_states.shape
    E, FF2, _ = gate_up_proj.shape

    W = _routing_weights(top_k_indices, top_k_weights, E)

    return pl.pallas_call(
        _moe_body,
        grid=(E // _EPB,),
        in_specs=[
            pl.BlockSpec((T, E), lambda e: (0, 0)),
            pl.BlockSpec((T, H), lambda e: (0, 0)),
            pl.BlockSpec((_EPB, FF2, H), lambda e: (e, 0, 0)),
            pl.BlockSpec((_EPB, H, FF2 // 2), lambda e: (e, 0, 0)),
        ],
        out_specs=pl.BlockSpec((T, H), lambda e: (0, 0)),
        out_shape=jax.ShapeDtypeStruct((T, H), jnp.float32),
        compiler_params=pltpu.CompilerParams(
            dimension_semantics=("arbitrary",),
        ),
    )(W, hidden_states, gate_up_proj, down_proj)


# speedup vs baseline: 1.0575x; 1.0133x over previous
import functools

import jax
import jax.numpy as jnp
from jax import lax
from jax.experimental import pallas as pl
from jax.experimental.pallas import tpu as pltpu
from jax.experimental.pallas import tpu_sc as plsc

_FF = 512
_EPB = 2


def _routing_body(T, E, K, NC, NS, idx_hbm, wgt_hbm, w_hbm, idx_v, wgt_v, wloc_v):
    NW = NC * NS
    tpw = T // NW
    vpw = tpw * K
    wid = lax.axis_index("s") * NC + lax.axis_index("c")
    pltpu.sync_copy(idx_hbm.at[pl.ds(wid * vpw, vpw)], idx_v)
    pltpu.sync_copy(wgt_hbm.at[pl.ds(wid * vpw, vpw)], wgt_v)
    for j in range(tpw * E // 16):
        wloc_v[pl.ds(j * 16, 16)] = jnp.zeros((16,), jnp.float32)
    lanes = lax.broadcasted_iota(jnp.int32, (16,), 0)
    for j in range(vpw // 16):
        local_t = (lanes + j * 16) // K
        flat = local_t * E + idx_v[pl.ds(j * 16, 16)]
        wv = wgt_v[pl.ds(j * 16, 16)]
        for k in range(K):
            plsc.addupdate_scatter(wloc_v, [flat], wv, mask=(lanes % K) == k)
    pltpu.sync_copy(wloc_v, w_hbm.at[pl.ds(wid * tpw * E, tpw * E)])


def _routing_weights(top_k_indices, top_k_weights, E):
    T, K = top_k_indices.shape
    info = plsc.get_sparse_core_info()
    NC, NS = 1, info.num_subcores
    NW = NC * NS
    tpw = T // NW
    mesh = plsc.VectorSubcoreMesh(core_axis_name="c", subcore_axis_name="s",
                                  num_cores=NC, num_subcores=NS)
    k = pl.kernel(
        functools.partial(_routing_body, T, E, K, NC, NS),
        out_type=jax.ShapeDtypeStruct((T * E,), jnp.float32),
        mesh=mesh,
        scratch_types=[
            pltpu.VMEM((tpw * K,), jnp.int32),
            pltpu.VMEM((tpw * K,), jnp.float32),
            pltpu.VMEM((tpw * E,), jnp.float32),
        ],
        compiler_params=pltpu.CompilerParams(needs_layout_passes=False),
    )
    w = k(top_k_indices.reshape(-1).astype(jnp.int32),
          top_k_weights.reshape(-1))
    return w.reshape(T, E)


def _moe_body(w_ref, hs_ref, gup_ref, down_ref, out_ref):
    g = pl.program_id(0)
    hs = hs_ref[...]
    W = w_ref[...]
    eidx = lax.broadcasted_iota(jnp.int32, W.shape, 1)
    contrib = jnp.zeros_like(out_ref)
    for i in range(_EPB):
        e = g * _EPB + i
        w = jnp.sum(jnp.where(eidx == e, W, 0.0), axis=1)
        gu = jax.lax.dot_general(
            hs, gup_ref[i], (((1,), (1,)), ((), ())),
            preferred_element_type=jnp.float32)
        gate = gu[:, :_FF]
        up = gu[:, _FF:]
        act = gate * jax.nn.sigmoid(gate) * up
        eo = jax.lax.dot_general(
            act, down_ref[i], (((1,), (1,)), ((), ())),
            preferred_element_type=jnp.float32)
        contrib = contrib + eo * w[:, None]

    @pl.when(g == 0)
    def _init():
        out_ref[...] = contrib

    @pl.when(g != 0)
    def _acc():
        out_ref[...] += contrib


def kernel(hidden_states, top_k_indices, top_k_weights, gate_up_proj, down_proj):
    T, H = hidden_states.shape
    E, FF2, _ = gate_up_proj.shape

    W = _routing_weights(top_k_indices, top_k_weights, E)

    return pl.pallas_call(
        _moe_body,
        grid=(E // _EPB,),
        in_specs=[
            pl.BlockSpec((T, E), lambda e: (0, 0)),
            pl.BlockSpec((T, H), lambda e: (0, 0)),
            pl.BlockSpec((_EPB, FF2, H), lambda e: (e, 0, 0)),
            pl.BlockSpec((_EPB, H, FF2 // 2), lambda e: (e, 0, 0)),
        ],
        out_specs=pl.BlockSpec((T, H), lambda e: (0, 0)),
        out_shape=jax.ShapeDtypeStruct((T, H), jnp.float32),
        compiler_params=pltpu.CompilerParams(
            dimension_semantics=("arbitrary",),
        ),
    )(W, hidden_states, gate_up_proj, down_proj)
